# baseline (speedup 1.0000x reference)
import jax
import jax.numpy as jnp
import numpy as np
from jax import lax
from jax.experimental import pallas as pl
from jax.experimental.pallas import tpu as pltpu

_B = 16
_T = 16
_DIN = 2048
_DSAE = 8192
_K = 128

_NT = 1024
_KT = 1024

_INT_MIN = np.int32(-2147483648)


def _enc_body(x_ref, w_ref, b_ref, o_ref):
    o_ref[...] = (
        jnp.dot(x_ref[...], w_ref[...], preferred_element_type=jnp.float32,
                precision=lax.Precision.DEFAULT)
        + b_ref[...]
    )


def _unsort_f(sv):
    bits = jnp.where(sv >= 0, sv, jnp.bitwise_not(jnp.bitwise_xor(sv, _INT_MIN)))
    return lax.bitcast_convert_type(bits, jnp.float32)


_CH = 8
_W = _DSAE // _CH


def _count3_rounds(buf, nrounds, kvec, bit0):
    def rnd(i, cur):
        step = lax.shift_left(jnp.int32(1), jnp.int32(bit0) - 2 * i)
        c1 = cur + step
        c2 = c1 + step
        c3 = c2 + step
        t1 = c1.astype(jnp.int16)
        t2 = c2.astype(jnp.int16)
        t3 = c3.astype(jnp.int16)
        a1 = jnp.zeros((_B, _W), jnp.int16)
        a2 = jnp.zeros((_B, _W), jnp.int16)
        a3 = jnp.zeros((_B, _W), jnp.int16)
        one = jnp.int16(1)
        zero = jnp.int16(0)
        for c in range(_CH):
            blk = buf[:, c * _W:(c + 1) * _W]
            a1 = a1 + jnp.where(blk >= t1, one, zero)
            a2 = a2 + jnp.where(blk >= t2, one, zero)
            a3 = a3 + jnp.where(blk >= t3, one, zero)
        n1 = jnp.sum(a1.astype(jnp.int32), axis=1, keepdims=True)
        n2 = jnp.sum(a2.astype(jnp.int32), axis=1, keepdims=True)
        n3 = jnp.sum(a3.astype(jnp.int32), axis=1, keepdims=True)
        m = ((n1 >= kvec).astype(jnp.int32) + (n2 >= kvec).astype(jnp.int32)
             + (n3 >= kvec).astype(jnp.int32))
        return cur + m * step

    return lax.fori_loop(0, nrounds, rnd, jnp.full((_B, 1), jnp.int32(-32768)))


def _rec_body(pre_ref, g_ref, z_ref, pbuf, hbuf, lbuf, tbuf, gbuf):
    gbuf[...] = 1.0 / (1.0 + jnp.exp(-g_ref[...]))
    kvec = jnp.full((_B, 1), jnp.int32(_K))
    for t in range(_T):
        base = t * _DSAE
        pbase = (t - 1) * _DSAE
        for c in range(_CH):
            lo = c * _W
            p = pre_ref[:, base + lo:base + lo + _W]
            if t > 0:
                p = p + gbuf[:, lo:lo + _W] * z_ref[:, pbase + lo:pbase + lo + _W]
            pbuf[:, lo:lo + _W] = p
            b = lax.bitcast_convert_type(p, jnp.int32)
            s = jnp.where(b >= 0, b,
                          jnp.bitwise_xor(jnp.bitwise_not(b), _INT_MIN))
            hbuf[:, lo:lo + _W] = lax.shift_right_arithmetic(s, 16).astype(jnp.int16)
            lbuf[:, lo:lo + _W] = jnp.bitwise_xor(s.astype(jnp.int16),
                                                  jnp.int16(-32768))

        cur_h = _count3_rounds(hbuf, 8, kvec, 14)
        th16 = cur_h.astype(jnp.int16)

        ag = jnp.zeros((_B, _W), jnp.int16)
        one = jnp.int16(1)
        zero = jnp.int16(0)
        neginf = jnp.int16(-32768)
        for c in range(_CH):
            lo = c * _W
            hblk = hbuf[:, lo:lo + _W]
            ag = ag + jnp.where(hblk > th16, one, zero)
            tbuf[:, lo:lo + _W] = jnp.where(hblk == th16,
                                            lbuf[:, lo:lo + _W], neginf)
        c_gt = jnp.sum(ag.astype(jnp.int32), axis=1, keepdims=True)

        cur_l = _count3_rounds(tbuf, 8, kvec - c_gt, 14)

        s_star = jnp.bitwise_or(
            lax.shift_left(cur_h, 16),
            jnp.bitwise_and(jnp.bitwise_xor(cur_l, jnp.int32(-32768)),
                            jnp.int32(0xFFFF)))
        thr = _unsort_f(s_star)
        for c in range(_CH):
            lo = c * _W
            p = pbuf[:, lo:lo + _W]
            z_ref[:, base + lo:base + lo + _W] = jnp.where(
                p >= thr, jnp.maximum(p, 0.0), 0.0)


def _dec_body(z_ref, w_ref, b_ref, x_ref, xh_ref, loss_ref):
    k = pl.program_id(0)
    part = jnp.dot(z_ref[...], w_ref[...], preferred_element_type=jnp.float32,
                   precision=lax.Precision.DEFAULT)

    @pl.when(k == 0)
    def _():
        xh_ref[...] = part

    @pl.when(k > 0)
    def _():
        xh_ref[...] += part

    @pl.when(k == pl.num_programs(0) - 1)
    def _():
        xh = xh_ref[...] + b_ref[...]
        xh_ref[...] = xh
        d = xh - x_ref[...]
        loss_ref[0, 0] = jnp.sum(d * d) * (1.0 / (_B * _T))


def kernel(x, W_enc, W_dec, b_enc, b_dec, gate_raw):
    x2 = x.reshape(_B * _T, _DIN)

    pre = pl.pallas_call(
        _enc_body,
        grid=(_DSAE // _NT,),
        in_specs=[
            pl.BlockSpec((_B * _T, _DIN), lambda n: (0, 0)),
            pl.BlockSpec((_DIN, _NT), lambda n: (0, n)),
            pl.BlockSpec((1, _NT), lambda n: (0, n)),
        ],
        out_specs=pl.BlockSpec((_B * _T, _NT), lambda n: (0, n)),
        out_shape=jax.ShapeDtypeStruct((_B * _T, _DSAE), jnp.float32),
        compiler_params=pltpu.CompilerParams(
            dimension_semantics=("parallel",)),
    )(x2, W_enc, b_enc.reshape(1, _DSAE))

    pre_b = pre.reshape(_B, _T * _DSAE)

    z = pl.pallas_call(
        _rec_body,
        in_specs=[
            pl.BlockSpec((_B, _T * _DSAE), lambda: (0, 0)),
            pl.BlockSpec((1, _DSAE), lambda: (0, 0)),
        ],
        out_specs=pl.BlockSpec((_B, _T * _DSAE), lambda: (0, 0)),
        out_shape=jax.ShapeDtypeStruct((_B, _T * _DSAE), jnp.float32),
        scratch_shapes=[
            pltpu.VMEM((_B, _DSAE), jnp.float32),
            pltpu.VMEM((_B, _DSAE), jnp.int16),
            pltpu.VMEM((_B, _DSAE), jnp.int16),
            pltpu.VMEM((_B, _DSAE), jnp.int16),
            pltpu.VMEM((1, _DSAE), jnp.float32),
        ],
    )(pre_b, gate_raw.reshape(1, _DSAE))

    z2 = z.reshape(_B * _T, _DSAE)

    xh, loss = pl.pallas_call(
        _dec_body,
        grid=(_DSAE // _KT,),
        in_specs=[
            pl.BlockSpec((_B * _T, _KT), lambda k: (0, k)),
            pl.BlockSpec((_KT, _DIN), lambda k: (k, 0)),
            pl.BlockSpec((1, _DIN), lambda k: (0, 0)),
            pl.BlockSpec((_B * _T, _DIN), lambda k: (0, 0)),
        ],
        out_specs=[
            pl.BlockSpec((_B * _T, _DIN), lambda k: (0, 0)),
            pl.BlockSpec(memory_space=pltpu.SMEM, block_shape=(1, 1),
                         index_map=lambda k: (0, 0)),
        ],
        out_shape=[
            jax.ShapeDtypeStruct((_B * _T, _DIN), jnp.float32),
            jax.ShapeDtypeStruct((1, 1), jnp.float32),
        ],
        compiler_params=pltpu.CompilerParams(
            dimension_semantics=("arbitrary",)),
    )(z2, W_dec, b_dec.reshape(1, _DIN), x2)

    x_hat = xh.reshape(_B, _T, _DIN)
    z_last = z[:, (_T - 1) * _DSAE:]
    return (loss[0, 0], x_hat, z_last)


# speedup vs baseline: 1.0060x; 1.0060x over previous
import jax
import jax.numpy as jnp
import numpy as np
from jax import lax
from jax.experimental import pallas as pl
from jax.experimental.pallas import tpu as pltpu

_B = 16
_T = 16
_DIN = 2048
_DSAE = 8192
_K = 128

_NT = 1024
_KT = 1024

_INT_MIN = np.int32(-2147483648)


def _enc_body(x_ref, w_ref, b_ref, o_ref):
    o_ref[...] = (
        jnp.dot(x_ref[...], w_ref[...], preferred_element_type=jnp.float32,
                precision=lax.Precision.DEFAULT)
        + b_ref[...]
    )


def _unsort_f(sv):
    bits = jnp.where(sv >= 0, sv, jnp.bitwise_not(jnp.bitwise_xor(sv, _INT_MIN)))
    return lax.bitcast_convert_type(bits, jnp.float32)


_CH = 16
_W = _DSAE // _CH


def _count3_rounds(buf, nrounds, kvec, bit0):
    def rnd(i, cur):
        step = lax.shift_left(jnp.int32(1), jnp.int32(bit0) - 2 * i)
        c1 = cur + step
        c2 = c1 + step
        c3 = c2 + step
        t1 = c1.astype(jnp.int16)
        t2 = c2.astype(jnp.int16)
        t3 = c3.astype(jnp.int16)
        a1 = jnp.zeros((_B, _W), jnp.int16)
        a2 = jnp.zeros((_B, _W), jnp.int16)
        a3 = jnp.zeros((_B, _W), jnp.int16)
        one = jnp.int16(1)
        zero = jnp.int16(0)
        for c in range(_CH):
            blk = buf[:, c * _W:(c + 1) * _W]
            a1 = a1 + jnp.where(blk >= t1, one, zero)
            a2 = a2 + jnp.where(blk >= t2, one, zero)
            a3 = a3 + jnp.where(blk >= t3, one, zero)
        n1 = jnp.sum(a1.astype(jnp.int32), axis=1, keepdims=True)
        n2 = jnp.sum(a2.astype(jnp.int32), axis=1, keepdims=True)
        n3 = jnp.sum(a3.astype(jnp.int32), axis=1, keepdims=True)
        m = ((n1 >= kvec).astype(jnp.int32) + (n2 >= kvec).astype(jnp.int32)
             + (n3 >= kvec).astype(jnp.int32))
        return cur + m * step

    return lax.fori_loop(0, nrounds, rnd, jnp.full((_B, 1), jnp.int32(-32768)))


def _rec_body(pre_ref, g_ref, z_ref, pbuf, hbuf, lbuf, tbuf, gbuf):
    gbuf[...] = 1.0 / (1.0 + jnp.exp(-g_ref[...]))
    kvec = jnp.full((_B, 1), jnp.int32(_K))
    for t in range(_T):
        base = t * _DSAE
        pbase = (t - 1) * _DSAE
        for c in range(_CH):
            lo = c * _W
            p = pre_ref[:, base + lo:base + lo + _W]
            if t > 0:
                p = p + gbuf[:, lo:lo + _W] * z_ref[:, pbase + lo:pbase + lo + _W]
            pbuf[:, lo:lo + _W] = p
            b = lax.bitcast_convert_type(p, jnp.int32)
            s = jnp.where(b >= 0, b,
                          jnp.bitwise_xor(jnp.bitwise_not(b), _INT_MIN))
            hbuf[:, lo:lo + _W] = lax.shift_right_arithmetic(s, 16).astype(jnp.int16)
            lbuf[:, lo:lo + _W] = jnp.bitwise_xor(s.astype(jnp.int16),
                                                  jnp.int16(-32768))

        cur_h = _count3_rounds(hbuf, 8, kvec, 14)
        th16 = cur_h.astype(jnp.int16)

        ag = jnp.zeros((_B, _W), jnp.int16)
        one = jnp.int16(1)
        zero = jnp.int16(0)
        neginf = jnp.int16(-32768)
        for c in range(_CH):
            lo = c * _W
            hblk = hbuf[:, lo:lo + _W]
            ag = ag + jnp.where(hblk > th16, one, zero)
            tbuf[:, lo:lo + _W] = jnp.where(hblk == th16,
                                            lbuf[:, lo:lo + _W], neginf)
        c_gt = jnp.sum(ag.astype(jnp.int32), axis=1, keepdims=True)

        cur_l = _count3_rounds(tbuf, 8, kvec - c_gt, 14)

        s_star = jnp.bitwise_or(
            lax.shift_left(cur_h, 16),
            jnp.bitwise_and(jnp.bitwise_xor(cur_l, jnp.int32(-32768)),
                            jnp.int32(0xFFFF)))
        thr = _unsort_f(s_star)
        for c in range(_CH):
            lo = c * _W
            p = pbuf[:, lo:lo + _W]
            z_ref[:, base + lo:base + lo + _W] = jnp.where(
                p >= thr, jnp.maximum(p, 0.0), 0.0)


def _dec_body(z_ref, w_ref, b_ref, x_ref, xh_ref, loss_ref):
    k = pl.program_id(0)
    part = jnp.dot(z_ref[...], w_ref[...], preferred_element_type=jnp.float32,
                   precision=lax.Precision.DEFAULT)

    @pl.when(k == 0)
    def _():
        xh_ref[...] = part

    @pl.when(k > 0)
    def _():
        xh_ref[...] += part

    @pl.when(k == pl.num_programs(0) - 1)
    def _():
        xh = xh_ref[...] + b_ref[...]
        xh_ref[...] = xh
        d = xh - x_ref[...]
        loss_ref[0, 0] = jnp.sum(d * d) * (1.0 / (_B * _T))


def kernel(x, W_enc, W_dec, b_enc, b_dec, gate_raw):
    x2 = x.reshape(_B * _T, _DIN)

    pre = pl.pallas_call(
        _enc_body,
        grid=(_DSAE // _NT,),
        in_specs=[
            pl.BlockSpec((_B * _T, _DIN), lambda n: (0, 0)),
            pl.BlockSpec((_DIN, _NT), lambda n: (0, n)),
            pl.BlockSpec((1, _NT), lambda n: (0, n)),
        ],
        out_specs=pl.BlockSpec((_B * _T, _NT), lambda n: (0, n)),
        out_shape=jax.ShapeDtypeStruct((_B * _T, _DSAE), jnp.float32),
        compiler_params=pltpu.CompilerParams(
            dimension_semantics=("parallel",)),
    )(x2, W_enc, b_enc.reshape(1, _DSAE))

    pre_b = pre.reshape(_B, _T * _DSAE)

    z = pl.pallas_call(
        _rec_body,
        in_specs=[
            pl.BlockSpec((_B, _T * _DSAE), lambda: (0, 0)),
            pl.BlockSpec((1, _DSAE), lambda: (0, 0)),
        ],
        out_specs=pl.BlockSpec((_B, _T * _DSAE), lambda: (0, 0)),
        out_shape=jax.ShapeDtypeStruct((_B, _T * _DSAE), jnp.float32),
        scratch_shapes=[
            pltpu.VMEM((_B, _DSAE), jnp.float32),
            pltpu.VMEM((_B, _DSAE), jnp.int16),
            pltpu.VMEM((_B, _DSAE), jnp.int16),
            pltpu.VMEM((_B, _DSAE), jnp.int16),
            pltpu.VMEM((1, _DSAE), jnp.float32),
        ],
    )(pre_b, gate_raw.reshape(1, _DSAE))

    z2 = z.reshape(_B * _T, _DSAE)

    xh, loss = pl.pallas_call(
        _dec_body,
        grid=(_DSAE // _KT,),
        in_specs=[
            pl.BlockSpec((_B * _T, _KT), lambda k: (0, k)),
            pl.BlockSpec((_KT, _DIN), lambda k: (k, 0)),
            pl.BlockSpec((1, _DIN), lambda k: (0, 0)),
            pl.BlockSpec((_B * _T, _DIN), lambda k: (0, 0)),
        ],
        out_specs=[
            pl.BlockSpec((_B * _T, _DIN), lambda k: (0, 0)),
            pl.BlockSpec(memory_space=pltpu.SMEM, block_shape=(1, 1),
                         index_map=lambda k: (0, 0)),
        ],
        out_shape=[
            jax.ShapeDtypeStruct((_B * _T, _DIN), jnp.float32),
            jax.ShapeDtypeStruct((1, 1), jnp.float32),
        ],
        compiler_params=pltpu.CompilerParams(
            dimension_semantics=("arbitrary",)),
    )(z2, W_dec, b_dec.reshape(1, _DIN), x2)

    x_hat = xh.reshape(_B, _T, _DIN)
    z_last = z[:, (_T - 1) * _DSAE:]
    return (loss[0, 0], x_hat, z_last)
